# baseline (speedup 1.0000x reference)
import functools

import jax
import jax.numpy as jnp
from jax import lax
from jax.experimental import pallas as pl
from jax.experimental.pallas import tpu as pltpu
from jax.experimental.pallas import tpu_sc as plsc

N = 10000
E = 320000
D = 128

NC = 2
NS = 16
GROUP = 128
GPW = 80
E_PAD = NC * NS * GPW * GROUP
N_ACC = 10240
RPT = N_ACC // NS

GPW0 = 160
GPW1 = 0


QB = 32


def _sc_seg_sum_body(want_deg, gpw0, gpw1, tab_hbm, src_hbm, dst_hbm, z2_hbm,
                     z1_hbm, o1_hbm, p0_hbm, p1_hbm, g0_hbm, g1_hbm,
                     acc_s, deg_s, sidx_v, didx_v, rows0_v, rows1_v, zero1_v,
                     ones_v, sem0, sem1):
    cid = lax.axis_index("c")
    sid = lax.axis_index("s")
    base = sid * RPT

    def init_tile():
        pltpu.sync_copy(z2_hbm, rows0_v)
        for k in range(RPT // 128):
            pltpu.sync_copy(rows0_v, acc_s.at[pl.ds(base + k * 128, 128)])
        if want_deg:
            pltpu.sync_copy(o1_hbm, ones_v)
            pltpu.sync_copy(z1_hbm, zero1_v)
            pltpu.sync_copy(zero1_v, deg_s.at[pl.ds(base, RPT)])

    if gpw1 == 0:
        @pl.when(cid == 0)
        def _():
            init_tile()
    else:
        init_tile()
    plsc.subcore_barrier()

    def gather(idx_row, rows_v, sem):
        pltpu.async_copy(tab_hbm.at[sidx_v.at[idx_row]], rows_v, sem)

    def drain(idx_row, rows_v, sem):
        pltpu.make_async_copy(tab_hbm.at[sidx_v.at[idx_row]], rows_v,
                              sem).wait()

    def scatter(idx_row, rows_v):
        pltpu.sync_copy(rows_v, acc_s.at[didx_v.at[idx_row]], add=True)
        if want_deg:
            pltpu.sync_copy(ones_v, deg_s.at[didx_v.at[idx_row]], add=True)

    def do_edges(gbase, nbatch):
        for h in range(nbatch):
            pltpu.sync_copy(src_hbm.at[pl.ds(gbase + h * QB, QB)], sidx_v)
            pltpu.sync_copy(dst_hbm.at[pl.ds(gbase + h * QB, QB)], didx_v)
            gather(0, rows0_v, sem0)

            def pair(i, carry):
                g0 = 2 * i
                gather(g0 + 1, rows1_v, sem1)
                drain(g0, rows0_v, sem0)
                scatter(g0, rows0_v)
                gather(g0 + 2, rows0_v, sem0)
                drain(g0 + 1, rows1_v, sem1)
                scatter(g0 + 1, rows1_v)
                return carry

            lax.fori_loop(0, QB // 2 - 1, pair, 0)
            gather(QB - 1, rows1_v, sem1)
            drain(QB - 2, rows0_v, sem0)
            scatter(QB - 2, rows0_v)
            drain(QB - 1, rows1_v, sem1)
            scatter(QB - 1, rows1_v)

    if gpw0:
        @pl.when(cid == 0)
        def _():
            do_edges(sid * gpw0, gpw0 // QB)

    if gpw1:
        @pl.when(cid == 1)
        def _():
            do_edges(NS * gpw0 + sid * gpw1, gpw1 // QB)

    plsc.subcore_barrier()

    @pl.when(cid == 0)
    def _():
        pltpu.sync_copy(acc_s.at[pl.ds(base, RPT)], p0_hbm.at[pl.ds(base, RPT)])
        if want_deg:
            pltpu.sync_copy(deg_s.at[pl.ds(base, RPT)],
                            g0_hbm.at[pl.ds(base, RPT)])

    if gpw1:
        @pl.when(cid == 1)
        def _():
            pltpu.sync_copy(acc_s.at[pl.ds(base, RPT)],
                            p1_hbm.at[pl.ds(base, RPT)])
            if want_deg:
                pltpu.sync_copy(deg_s.at[pl.ds(base, RPT)],
                                g1_hbm.at[pl.ds(base, RPT)])


def _sc_out_type(want_deg, gpw1):
    mat = jax.ShapeDtypeStruct((N_ACC, D), jnp.float32)
    vec = jax.ShapeDtypeStruct((N_ACC,), jnp.float32)
    out = [mat]
    if gpw1:
        out.append(mat)
    if want_deg:
        out.append(vec)
        if gpw1:
            out.append(vec)
    return out


def _sc_body_with_outputs(want_deg, gpw0, gpw1):
    n_out = len(_sc_out_type(want_deg, gpw1))

    def body(*refs):
        ins = refs[:6]
        outs = list(refs[6:6 + n_out])
        scratch = refs[6 + n_out:]
        p0 = outs.pop(0)
        p1 = outs.pop(0) if gpw1 else None
        g0 = outs.pop(0) if want_deg else None
        g1 = outs.pop(0) if (want_deg and gpw1) else None
        return _sc_seg_sum_body(want_deg, gpw0, gpw1, *ins, p0, p1, g0, g1,
                                *scratch)

    return body


@functools.cache
def _sc_seg_sum(want_deg, gpw0=GPW, gpw1=GPW):
    mesh = plsc.VectorSubcoreMesh(core_axis_name="c", subcore_axis_name="s",
                                  num_cores=NC, num_subcores=NS)
    return pl.kernel(
        _sc_body_with_outputs(want_deg, gpw0, gpw1),
        out_type=_sc_out_type(want_deg, gpw1),
        mesh=mesh,
        scratch_types=[
            pltpu.VMEM_SHARED((N_ACC, D), jnp.float32),
            pltpu.VMEM_SHARED((N_ACC,), jnp.float32),
            pltpu.VMEM((QB, GROUP), jnp.int32),
            pltpu.VMEM((QB, GROUP), jnp.int32),
            pltpu.VMEM((GROUP, D), jnp.float32),
            pltpu.VMEM((GROUP, D), jnp.float32),
            pltpu.VMEM((RPT,), jnp.float32),
            pltpu.VMEM((GROUP,), jnp.float32),
            pltpu.SemaphoreType.DMA,
            pltpu.SemaphoreType.DMA,
        ],
    )


def _tc_layer_body(relu, two, *refs):
    if two:
        h_ref, p0_ref, p1_ref, d0_ref, d1_ref, ws_ref, wn_ref, b_ref, o_ref \
            = refs
        psum = p0_ref[...] + p1_ref[...]
        dsum = d0_ref[...] + d1_ref[...]
    else:
        h_ref, p0_ref, d0_ref, ws_ref, wn_ref, b_ref, o_ref = refs
        psum = p0_ref[...]
        dsum = d0_ref[...]
    agg = psum * (1.0 / jnp.maximum(dsum, 1.0))
    o = jnp.dot(h_ref[...], ws_ref[...], preferred_element_type=jnp.float32)
    o = o + jnp.dot(agg, wn_ref[...], preferred_element_type=jnp.float32)
    o = o + b_ref[...]
    if relu:
        o = jnp.maximum(o, 0.0)
    o_ref[...] = o


@functools.cache
def _tc_layer(relu, two):
    blk = 400
    grid = N // blk
    mat = pl.BlockSpec((blk, D), lambda i: (i, 0))
    vec = pl.BlockSpec((blk, 1), lambda i: (i, 0))
    wspec = pl.BlockSpec((D, D), lambda i: (0, 0))
    bspec = pl.BlockSpec((1, D), lambda i: (0, 0))
    if two:
        in_specs = [mat, mat, mat, vec, vec, wspec, wspec, bspec]
    else:
        in_specs = [mat, mat, vec, wspec, wspec, bspec]
    return pl.pallas_call(
        functools.partial(_tc_layer_body, relu, two),
        grid=(grid,),
        in_specs=in_specs,
        out_specs=pl.BlockSpec((blk, D), lambda i: (i, 0)),
        out_shape=jax.ShapeDtypeStruct((N, D), jnp.float32),
    )


def kernel(x, edge_index, W_self1, W_neigh1, b1, W_self2, W_neigh2, b2):
    src = edge_index[0]
    dst = edge_index[1]
    padv = jnp.full((E_PAD - E,), N, dtype=jnp.int32)
    src2 = jnp.concatenate([src, padv]).reshape(E_PAD // GROUP, GROUP)
    dst2 = jnp.concatenate([dst, padv]).reshape(E_PAD // GROUP, GROUP)
    zrow = jnp.zeros((8, D), jnp.float32)
    z2 = jnp.zeros((128, D), jnp.float32)
    z1 = jnp.zeros((RPT,), jnp.float32)
    o1 = jnp.ones((GROUP,), jnp.float32)

    xt = jnp.concatenate([x, zrow], axis=0)
    two = GPW1 > 0
    if two:
        p0, p1, g0, g1 = _sc_seg_sum(True, GPW0, GPW1)(xt, src2, dst2,
                                                       z2, z1, o1)
        h = _tc_layer(True, True)(x, p0, p1, g0[:, None], g1[:, None],
                                  W_self1, W_neigh1, b1.reshape(1, D))
    else:
        p0, g0 = _sc_seg_sum(True, GPW0, GPW1)(xt, src2, dst2, z2, z1, o1)
        h = _tc_layer(True, False)(x, p0, g0[:, None],
                                   W_self1, W_neigh1, b1.reshape(1, D))

    ht = jnp.concatenate([h, zrow], axis=0)
    if two:
        q0, q1 = _sc_seg_sum(False, GPW0, GPW1)(ht, src2, dst2, z2, z1, o1)
        out = _tc_layer(False, True)(h, q0, q1, g0[:, None], g1[:, None],
                                     W_self2, W_neigh2, b2.reshape(1, D))
    else:
        q0, = _sc_seg_sum(False, GPW0, GPW1)(ht, src2, dst2, z2, z1, o1)
        out = _tc_layer(False, False)(h, q0, g0[:, None],
                                      W_self2, W_neigh2, b2.reshape(1, D))
    return out


# speedup vs baseline: 2.9108x; 2.9108x over previous
import functools

import jax
import jax.numpy as jnp
from jax import lax
from jax.experimental import pallas as pl
from jax.experimental.pallas import tpu as pltpu
from jax.experimental.pallas import tpu_sc as plsc

N = 10000
E = 320000
D = 128

NC = 2
NS = 16
GROUP = 128
GPW = 80
E_PAD = NC * NS * GPW * GROUP
N_ACC = 10240
RPT = N_ACC // NS

GPW0 = 80
GPW1 = 80


QB = 16


def _sc_seg_sum_body(want_deg, gpw0, gpw1, tab_hbm, src_hbm, dst_hbm, z2_hbm,
                     z1_hbm, o1_hbm, p0_hbm, p1_hbm, g0_hbm, g1_hbm,
                     acc_s, deg_s, sidx_v, didx_v, rows0_v, rows1_v, zero1_v,
                     ones_v, sem0, sem1):
    cid = lax.axis_index("c")
    sid = lax.axis_index("s")
    base = sid * RPT

    def init_tile():
        pltpu.sync_copy(z2_hbm, rows0_v)
        for k in range(RPT // 128):
            pltpu.sync_copy(rows0_v, acc_s.at[pl.ds(base + k * 128, 128)])
        if want_deg:
            pltpu.sync_copy(o1_hbm, ones_v)
            pltpu.sync_copy(z1_hbm, zero1_v)
            pltpu.sync_copy(zero1_v, deg_s.at[pl.ds(base, RPT)])

    if gpw1 == 0:
        @pl.when(cid == 0)
        def _():
            init_tile()
    else:
        init_tile()
    plsc.subcore_barrier()

    def gather(idx_row, rows_v, sem):
        pltpu.async_copy(tab_hbm.at[sidx_v.at[idx_row]], rows_v, sem)

    def drain(idx_row, rows_v, sem):
        pltpu.make_async_copy(tab_hbm.at[sidx_v.at[idx_row]], rows_v,
                              sem).wait()

    def scatter(idx_row, rows_v):
        pltpu.sync_copy(rows_v, acc_s.at[didx_v.at[idx_row]], add=True)
        if want_deg:
            pltpu.sync_copy(ones_v, deg_s.at[didx_v.at[idx_row]], add=True)

    def do_edges(gbase, nbatch):
        assert nbatch * QB in (gpw0, gpw1)
        for h in range(nbatch):
            pltpu.sync_copy(src_hbm.at[pl.ds(gbase + h * QB, QB)], sidx_v)
            pltpu.sync_copy(dst_hbm.at[pl.ds(gbase + h * QB, QB)], didx_v)
            gather(0, rows0_v, sem0)

            def pair(i, carry):
                g0 = 2 * i
                gather(g0 + 1, rows1_v, sem1)
                drain(g0, rows0_v, sem0)
                scatter(g0, rows0_v)
                gather(g0 + 2, rows0_v, sem0)
                drain(g0 + 1, rows1_v, sem1)
                scatter(g0 + 1, rows1_v)
                return carry

            lax.fori_loop(0, QB // 2 - 1, pair, 0)
            gather(QB - 1, rows1_v, sem1)
            drain(QB - 2, rows0_v, sem0)
            scatter(QB - 2, rows0_v)
            drain(QB - 1, rows1_v, sem1)
            scatter(QB - 1, rows1_v)

    if gpw0:
        @pl.when(cid == 0)
        def _():
            do_edges(sid * gpw0, gpw0 // QB)

    if gpw1:
        @pl.when(cid == 1)
        def _():
            do_edges(NS * gpw0 + sid * gpw1, gpw1 // QB)

    plsc.subcore_barrier()

    @pl.when(cid == 0)
    def _():
        pltpu.sync_copy(acc_s.at[pl.ds(base, RPT)], p0_hbm.at[pl.ds(base, RPT)])
        if want_deg:
            pltpu.sync_copy(deg_s.at[pl.ds(base, RPT)],
                            g0_hbm.at[pl.ds(base, RPT)])

    if gpw1:
        @pl.when(cid == 1)
        def _():
            pltpu.sync_copy(acc_s.at[pl.ds(base, RPT)],
                            p1_hbm.at[pl.ds(base, RPT)])
            if want_deg:
                pltpu.sync_copy(deg_s.at[pl.ds(base, RPT)],
                                g1_hbm.at[pl.ds(base, RPT)])


def _sc_out_type(want_deg, gpw1):
    mat = jax.ShapeDtypeStruct((N_ACC, D), jnp.float32)
    vec = jax.ShapeDtypeStruct((N_ACC,), jnp.float32)
    out = [mat]
    if gpw1:
        out.append(mat)
    if want_deg:
        out.append(vec)
        if gpw1:
            out.append(vec)
    return out


def _sc_body_with_outputs(want_deg, gpw0, gpw1):
    n_out = len(_sc_out_type(want_deg, gpw1))

    def body(*refs):
        ins = refs[:6]
        outs = list(refs[6:6 + n_out])
        scratch = refs[6 + n_out:]
        p0 = outs.pop(0)
        p1 = outs.pop(0) if gpw1 else None
        g0 = outs.pop(0) if want_deg else None
        g1 = outs.pop(0) if (want_deg and gpw1) else None
        return _sc_seg_sum_body(want_deg, gpw0, gpw1, *ins, p0, p1, g0, g1,
                                *scratch)

    return body


@functools.cache
def _sc_seg_sum(want_deg, gpw0=GPW, gpw1=GPW):
    mesh = plsc.VectorSubcoreMesh(core_axis_name="c", subcore_axis_name="s",
                                  num_cores=NC, num_subcores=NS)
    return pl.kernel(
        _sc_body_with_outputs(want_deg, gpw0, gpw1),
        out_type=_sc_out_type(want_deg, gpw1),
        mesh=mesh,
        scratch_types=[
            pltpu.VMEM_SHARED((N_ACC, D), jnp.float32),
            pltpu.VMEM_SHARED((N_ACC,), jnp.float32),
            pltpu.VMEM((QB, GROUP), jnp.int32),
            pltpu.VMEM((QB, GROUP), jnp.int32),
            pltpu.VMEM((GROUP, D), jnp.float32),
            pltpu.VMEM((GROUP, D), jnp.float32),
            pltpu.VMEM((RPT,), jnp.float32),
            pltpu.VMEM((GROUP,), jnp.float32),
            pltpu.SemaphoreType.DMA,
            pltpu.SemaphoreType.DMA,
        ],
    )


def _tc_layer_body(relu, two, *refs):
    if two:
        h_ref, p0_ref, p1_ref, d0_ref, d1_ref, ws_ref, wn_ref, b_ref, o_ref \
            = refs
        psum = p0_ref[...] + p1_ref[...]
        dsum = d0_ref[...] + d1_ref[...]
    else:
        h_ref, p0_ref, d0_ref, ws_ref, wn_ref, b_ref, o_ref = refs
        psum = p0_ref[...]
        dsum = d0_ref[...]
    agg = psum * (1.0 / jnp.maximum(dsum, 1.0))
    o = jnp.dot(h_ref[...], ws_ref[...], preferred_element_type=jnp.float32)
    o = o + jnp.dot(agg, wn_ref[...], preferred_element_type=jnp.float32)
    o = o + b_ref[...]
    if relu:
        o = jnp.maximum(o, 0.0)
    o_ref[...] = o


@functools.cache
def _tc_layer(relu, two):
    blk = 400
    grid = N // blk
    mat = pl.BlockSpec((blk, D), lambda i: (i, 0))
    vec = pl.BlockSpec((blk, 1), lambda i: (i, 0))
    wspec = pl.BlockSpec((D, D), lambda i: (0, 0))
    bspec = pl.BlockSpec((1, D), lambda i: (0, 0))
    if two:
        in_specs = [mat, mat, mat, vec, vec, wspec, wspec, bspec]
    else:
        in_specs = [mat, mat, vec, wspec, wspec, bspec]
    return pl.pallas_call(
        functools.partial(_tc_layer_body, relu, two),
        grid=(grid,),
        in_specs=in_specs,
        out_specs=pl.BlockSpec((blk, D), lambda i: (i, 0)),
        out_shape=jax.ShapeDtypeStruct((N, D), jnp.float32),
    )


def kernel(x, edge_index, W_self1, W_neigh1, b1, W_self2, W_neigh2, b2):
    src = edge_index[0]
    dst = edge_index[1]
    npad = E_PAD - E
    pad_src = N + (jnp.arange(npad, dtype=jnp.int32) % 8)
    pad_dst = N + (jnp.arange(npad, dtype=jnp.int32) % (N_ACC - N))
    src2 = jnp.concatenate([src, pad_src]).reshape(E_PAD // GROUP, GROUP)
    dst2 = jnp.concatenate([dst, pad_dst]).reshape(E_PAD // GROUP, GROUP)
    zrow = jnp.zeros((8, D), jnp.float32)
    z2 = jnp.zeros((128, D), jnp.float32)
    z1 = jnp.zeros((RPT,), jnp.float32)
    o1 = jnp.ones((GROUP,), jnp.float32)

    xt = jnp.concatenate([x, zrow], axis=0)
    two = GPW1 > 0
    if two:
        p0, p1, g0, g1 = _sc_seg_sum(True, GPW0, GPW1)(xt, src2, dst2,
                                                       z2, z1, o1)
        h = _tc_layer(True, True)(x, p0, p1, g0[:, None], g1[:, None],
                                  W_self1, W_neigh1, b1.reshape(1, D))
    else:
        p0, g0 = _sc_seg_sum(True, GPW0, GPW1)(xt, src2, dst2, z2, z1, o1)
        h = _tc_layer(True, False)(x, p0, g0[:, None],
                                   W_self1, W_neigh1, b1.reshape(1, D))

    ht = jnp.concatenate([h, zrow], axis=0)
    if two:
        q0, q1 = _sc_seg_sum(False, GPW0, GPW1)(ht, src2, dst2, z2, z1, o1)
        out = _tc_layer(False, True)(h, q0, q1, g0[:, None], g1[:, None],
                                     W_self2, W_neigh2, b2.reshape(1, D))
    else:
        q0, = _sc_seg_sum(False, GPW0, GPW1)(ht, src2, dst2, z2, z1, o1)
        out = _tc_layer(False, False)(h, q0, g0[:, None],
                                      W_self2, W_neigh2, b2.reshape(1, D))
    return out


# speedup vs baseline: 3.4439x; 1.1831x over previous
import functools

import jax
import jax.numpy as jnp
from jax import lax
from jax.experimental import pallas as pl
from jax.experimental.pallas import tpu as pltpu
from jax.experimental.pallas import tpu_sc as plsc

N = 10000
E = 320000
D = 128

NC = 2
NS = 16
GROUP = 128
GPW = 80
E_PAD = NC * NS * GPW * GROUP
N_ACC = 10240
RPT = N_ACC // NS

GPW0 = 80
GPW1 = 80


QB = 16


def _sc_seg_sum_body(want_deg, gpw0, gpw1, tab_hbm, src_hbm, dst_hbm, z2_hbm,
                     z1_hbm, o1_hbm, p0_hbm, p1_hbm, g0_hbm, g1_hbm,
                     acc_s, deg_s, sidx_v, didx_v, rows0_v, rows1_v, zero1_v,
                     ones_v, sem0, sem1):
    cid = lax.axis_index("c")
    sid = lax.axis_index("s")
    base = sid * RPT

    def init_tile():
        pltpu.sync_copy(z2_hbm, rows0_v)
        for k in range(RPT // 128):
            pltpu.sync_copy(rows0_v, acc_s.at[pl.ds(base + k * 128, 128)])
        if want_deg:
            pltpu.sync_copy(o1_hbm, ones_v)
            pltpu.sync_copy(z1_hbm, zero1_v)
            pltpu.sync_copy(zero1_v, deg_s.at[pl.ds(base, RPT)])

    if gpw1 == 0:
        @pl.when(cid == 0)
        def _():
            init_tile()
    else:
        init_tile()
    plsc.subcore_barrier()

    def gather(idx_row, rows_v, sem):
        pltpu.async_copy(tab_hbm.at[sidx_v.at[idx_row]], rows_v, sem)

    def drain(idx_row, rows_v, sem):
        pltpu.make_async_copy(tab_hbm.at[sidx_v.at[idx_row]], rows_v,
                              sem).wait()

    def scatter(idx_row, rows_v):
        pltpu.sync_copy(rows_v, acc_s.at[didx_v.at[idx_row]], add=True)
        if want_deg:
            pltpu.sync_copy(ones_v, deg_s.at[didx_v.at[idx_row]], add=True)

    def do_edges(gbase, nbatch):
        assert nbatch * QB in (gpw0, gpw1)
        for h in range(nbatch):
            pltpu.sync_copy(src_hbm.at[pl.ds(gbase + h * QB, QB)], sidx_v)
            pltpu.sync_copy(dst_hbm.at[pl.ds(gbase + h * QB, QB)], didx_v)
            gather(0, rows0_v, sem0)

            def pair(i, carry):
                g0 = 2 * i
                gather(g0 + 1, rows1_v, sem1)
                drain(g0, rows0_v, sem0)
                scatter(g0, rows0_v)
                gather(g0 + 2, rows0_v, sem0)
                drain(g0 + 1, rows1_v, sem1)
                scatter(g0 + 1, rows1_v)
                return carry

            lax.fori_loop(0, QB // 2 - 1, pair, 0)
            gather(QB - 1, rows1_v, sem1)
            drain(QB - 2, rows0_v, sem0)
            scatter(QB - 2, rows0_v)
            drain(QB - 1, rows1_v, sem1)
            scatter(QB - 1, rows1_v)

    if gpw0:
        @pl.when(cid == 0)
        def _():
            do_edges(sid * gpw0, gpw0 // QB)

    if gpw1:
        @pl.when(cid == 1)
        def _():
            do_edges(NS * gpw0 + sid * gpw1, gpw1 // QB)

    plsc.subcore_barrier()

    @pl.when(cid == 0)
    def _():
        pltpu.sync_copy(acc_s.at[pl.ds(base, RPT)], p0_hbm.at[pl.ds(base, RPT)])
        if want_deg:
            pltpu.sync_copy(deg_s.at[pl.ds(base, RPT)],
                            g0_hbm.at[pl.ds(base, RPT)])

    if gpw1:
        @pl.when(cid == 1)
        def _():
            pltpu.sync_copy(acc_s.at[pl.ds(base, RPT)],
                            p1_hbm.at[pl.ds(base, RPT)])
            if want_deg:
                pltpu.sync_copy(deg_s.at[pl.ds(base, RPT)],
                                g1_hbm.at[pl.ds(base, RPT)])


def _sc_out_type(want_deg, gpw1):
    mat = jax.ShapeDtypeStruct((N_ACC, D), jnp.float32)
    vec = jax.ShapeDtypeStruct((N_ACC,), jnp.float32)
    out = [mat]
    if gpw1:
        out.append(mat)
    if want_deg:
        out.append(vec)
        if gpw1:
            out.append(vec)
    return out


def _sc_body_with_outputs(want_deg, gpw0, gpw1):
    n_out = len(_sc_out_type(want_deg, gpw1))

    def body(*refs):
        ins = refs[:6]
        outs = list(refs[6:6 + n_out])
        scratch = refs[6 + n_out:]
        p0 = outs.pop(0)
        p1 = outs.pop(0) if gpw1 else None
        g0 = outs.pop(0) if want_deg else None
        g1 = outs.pop(0) if (want_deg and gpw1) else None
        return _sc_seg_sum_body(want_deg, gpw0, gpw1, *ins, p0, p1, g0, g1,
                                *scratch)

    return body


@functools.cache
def _sc_seg_sum(want_deg, gpw0=GPW, gpw1=GPW):
    mesh = plsc.VectorSubcoreMesh(core_axis_name="c", subcore_axis_name="s",
                                  num_cores=NC, num_subcores=NS)
    return pl.kernel(
        _sc_body_with_outputs(want_deg, gpw0, gpw1),
        out_type=_sc_out_type(want_deg, gpw1),
        mesh=mesh,
        scratch_types=[
            pltpu.VMEM_SHARED((N_ACC, D), jnp.float32),
            pltpu.VMEM_SHARED((N_ACC,), jnp.float32),
            pltpu.VMEM((QB, GROUP), jnp.int32),
            pltpu.VMEM((QB, GROUP), jnp.int32),
            pltpu.VMEM((GROUP, D), jnp.float32),
            pltpu.VMEM((GROUP, D), jnp.float32),
            pltpu.VMEM((RPT,), jnp.float32),
            pltpu.VMEM((GROUP,), jnp.float32),
            pltpu.SemaphoreType.DMA,
            pltpu.SemaphoreType.DMA,
        ],
    )


def _tc_layer_body(relu, two, *refs):
    if two:
        h_ref, p0_ref, p1_ref, d0_ref, d1_ref, ws_ref, wn_ref, b_ref, o_ref \
            = refs
        psum = p0_ref[...] + p1_ref[...]
        dsum = d0_ref[...] + d1_ref[...]
    else:
        h_ref, p0_ref, d0_ref, ws_ref, wn_ref, b_ref, o_ref = refs
        psum = p0_ref[...]
        dsum = d0_ref[...]
    agg = psum * (1.0 / jnp.maximum(dsum, 1.0))
    o = jnp.dot(h_ref[...], ws_ref[...], preferred_element_type=jnp.float32)
    o = o + jnp.dot(agg, wn_ref[...], preferred_element_type=jnp.float32)
    o = o + b_ref[...]
    if relu:
        o = jnp.maximum(o, 0.0)
    o_ref[...] = o


@functools.cache
def _tc_layer(relu, two):
    blk = 400
    grid = N // blk
    mat = pl.BlockSpec((blk, D), lambda i: (i, 0))
    vec = pl.BlockSpec((blk, 1), lambda i: (i, 0))
    wspec = pl.BlockSpec((D, D), lambda i: (0, 0))
    bspec = pl.BlockSpec((1, D), lambda i: (0, 0))
    if two:
        in_specs = [mat, mat, mat, vec, vec, wspec, wspec, bspec]
    else:
        in_specs = [mat, mat, vec, wspec, wspec, bspec]
    return pl.pallas_call(
        functools.partial(_tc_layer_body, relu, two),
        grid=(grid,),
        in_specs=in_specs,
        out_specs=pl.BlockSpec((blk, D), lambda i: (i, 0)),
        out_shape=jax.ShapeDtypeStruct((N, D), jnp.float32),
    )


def kernel(x, edge_index, W_self1, W_neigh1, b1, W_self2, W_neigh2, b2):
    src = edge_index[0]
    dst = edge_index[1]
    npad = E_PAD - E
    ar = jnp.arange(npad, dtype=jnp.int32)
    pad_src = (ar * 131) % N
    pad_dst = N + ar % (N_ACC - N)
    src2 = jnp.concatenate([src, pad_src]).reshape(E_PAD // GROUP, GROUP)
    dst2 = jnp.concatenate([dst, pad_dst]).reshape(E_PAD // GROUP, GROUP)
    z2 = jnp.zeros((128, D), jnp.float32)
    z1 = jnp.zeros((RPT,), jnp.float32)
    o1 = jnp.ones((GROUP,), jnp.float32)

    two = GPW1 > 0
    if two:
        p0, p1, g0, g1 = _sc_seg_sum(True, GPW0, GPW1)(x, src2, dst2,
                                                       z2, z1, o1)
        h = _tc_layer(True, True)(x, p0, p1, g0[:, None], g1[:, None],
                                  W_self1, W_neigh1, b1.reshape(1, D))
    else:
        p0, g0 = _sc_seg_sum(True, GPW0, GPW1)(x, src2, dst2, z2, z1, o1)
        h = _tc_layer(True, False)(x, p0, g0[:, None],
                                   W_self1, W_neigh1, b1.reshape(1, D))

    if two:
        q0, q1 = _sc_seg_sum(False, GPW0, GPW1)(h, src2, dst2, z2, z1, o1)
        out = _tc_layer(False, True)(h, q0, q1, g0[:, None], g1[:, None],
                                     W_self2, W_neigh2, b2.reshape(1, D))
    else:
        q0, = _sc_seg_sum(False, GPW0, GPW1)(h, src2, dst2, z2, z1, o1)
        out = _tc_layer(False, False)(h, q0, g0[:, None],
                                      W_self2, W_neigh2, b2.reshape(1, D))
    return out
